# baseline (speedup 1.0000x reference)
import dataclasses
import functools

import jax
import jax.numpy as jnp
from jax import lax
from jax.experimental import pallas as pl
from jax.experimental.pallas import tpu as pltpu
from jax.experimental.pallas import tpu_sc as plsc

B = 16384
NUM_CORES = 1
NUM_SUBCORES = 16
LANES = 16
NUM_WORKERS = NUM_CORES * NUM_SUBCORES
CHUNK = B // NUM_WORKERS

_COMPILER_PARAMS = pltpu.CompilerParams()
if "needs_layout_passes" in pltpu.CompilerParams.__dataclass_fields__:
    _COMPILER_PARAMS = dataclasses.replace(
        _COMPILER_PARAMS, needs_layout_passes=False)

_MESH = plsc.VectorSubcoreMesh(
    core_axis_name="c", subcore_axis_name="s",
    num_cores=NUM_CORES, num_subcores=NUM_SUBCORES,
)


@functools.partial(
    pl.kernel,
    out_type=jax.ShapeDtypeStruct((B,), jnp.float32),
    mesh=_MESH,
    scratch_types=[
        pltpu.VMEM((CHUNK,), jnp.int32),
        pltpu.VMEM((CHUNK,), jnp.float32),
        pltpu.VMEM((LANES,), jnp.float32),
        pltpu.SemaphoreType.DMA,
        pltpu.SemaphoreType.DMA,
    ],
    compiler_params=_COMPILER_PARAMS,
)
def _sc_gather(cs_hbm, p_hbm, out_hbm, idx_v, out_v, p_v, sem_i, sem_p):
    wid = lax.axis_index("s") * NUM_CORES + lax.axis_index("c")
    base = wid * CHUNK

    cp_p = pltpu.async_copy(p_hbm, p_v.at[pl.ds(0, 2)], sem_p)
    cp_i = pltpu.async_copy(cs_hbm.at[pl.ds(base, CHUNK)], idx_v, sem_i)
    cp_p.wait()
    pv_raw = p_v[pl.ds(0, LANES)]
    lane = lax.iota(jnp.int32, LANES)
    pv0 = jnp.sum(jnp.where(lane == 0, pv_raw, jnp.float32(0)))
    pv1 = jnp.sum(jnp.where(lane == 1, pv_raw, jnp.float32(0)))
    cp_i.wait()
    half = CHUNK // 2

    @plsc.parallel_loop(0, half, step=LANES, unroll=8)
    def _(i):
        iv = idx_v[pl.ds(i, LANES)]
        out_v[pl.ds(i, LANES)] = jnp.where(iv == 0, pv0, pv1)

    cp_o = pltpu.async_copy(
        out_v.at[pl.ds(0, half)], out_hbm.at[pl.ds(base, half)], sem_p)

    @plsc.parallel_loop(half, CHUNK, step=LANES, unroll=8)
    def _(i):
        iv = idx_v[pl.ds(i, LANES)]
        out_v[pl.ds(i, LANES)] = jnp.where(iv == 0, pv0, pv1)

    pltpu.sync_copy(
        out_v.at[pl.ds(half, half)], out_hbm.at[pl.ds(base + half, half)])
    cp_o.wait()


@jax.jit
def kernel(cs, xs, params):
    del xs
    return _sc_gather(cs.astype(jnp.int32), jnp.reshape(params, (-1,)))


# speedup vs baseline: 1.0100x; 1.0100x over previous
import dataclasses
import functools

import jax
import jax.numpy as jnp
from jax import lax
from jax.experimental import pallas as pl
from jax.experimental.pallas import tpu as pltpu
from jax.experimental.pallas import tpu_sc as plsc

B = 16384
NUM_CORES = 1
NUM_SUBCORES = 16
LANES = 16
NUM_WORKERS = NUM_CORES * NUM_SUBCORES
CHUNK = B // NUM_WORKERS

_COMPILER_PARAMS = pltpu.CompilerParams()
if "needs_layout_passes" in pltpu.CompilerParams.__dataclass_fields__:
    _COMPILER_PARAMS = dataclasses.replace(
        _COMPILER_PARAMS, needs_layout_passes=False)

_MESH = plsc.VectorSubcoreMesh(
    core_axis_name="c", subcore_axis_name="s",
    num_cores=NUM_CORES, num_subcores=NUM_SUBCORES,
)


@functools.partial(
    pl.kernel,
    out_type=jax.ShapeDtypeStruct((B,), jnp.float32),
    mesh=_MESH,
    scratch_types=[
        pltpu.VMEM((CHUNK,), jnp.int32),
        pltpu.VMEM((CHUNK,), jnp.float32),
        pltpu.VMEM((LANES,), jnp.float32),
        pltpu.SemaphoreType.DMA,
        pltpu.SemaphoreType.DMA,
    ],
    compiler_params=_COMPILER_PARAMS,
)
def _sc_gather(cs_hbm, p_hbm, out_hbm, idx_v, out_v, p_v, sem_i, sem_p):
    wid = lax.axis_index("s") * NUM_CORES + lax.axis_index("c")
    base = wid * CHUNK

    cp_p = pltpu.async_copy(p_hbm, p_v.at[pl.ds(0, 2)], sem_p)
    cp_i = pltpu.async_copy(cs_hbm.at[pl.ds(base, CHUNK)], idx_v, sem_i)
    cp_p.wait()
    pv_raw = p_v[pl.ds(0, LANES)]
    lane = lax.iota(jnp.int32, LANES)
    pv0 = jnp.sum(jnp.where(lane == 0, pv_raw, jnp.float32(0)))
    pv1 = jnp.sum(jnp.where(lane == 1, pv_raw, jnp.float32(0)))
    cp_i.wait()

    @plsc.parallel_loop(0, CHUNK, step=LANES, unroll=4)
    def _(i):
        iv = idx_v[pl.ds(i, LANES)]
        out_v[pl.ds(i, LANES)] = jnp.where(iv == 0, pv0, pv1)

    pltpu.sync_copy(out_v, out_hbm.at[pl.ds(base, CHUNK)])


@jax.jit
def kernel(cs, xs, params):
    del xs
    return _sc_gather(cs.astype(jnp.int32), jnp.reshape(params, (-1,)))
